# baseline (speedup 1.0000x reference)
import functools

import jax
import jax.numpy as jnp
from jax import lax
from jax.experimental import pallas as pl
from jax.experimental.pallas import tpu as pltpu
from jax.experimental.pallas import tpu_sc as plsc

_BETA = 5.5
_LOGIT_SCALE = 100.0
_C, _M, _D = 1000, 50, 1024
_RP = _M + 2
_NW = 32
_CPW = 32
_NCH = _D // 16
_BLOCKS = [(0, 8), (8, 8), (16, 8), (24, 8), (32, 8), (40, 8), (48, 4)]


def _bsum(v):
    return jnp.broadcast_to(jnp.sum(v), (16,))


def _rsqrt(x):
    i = lax.bitcast_convert_type(x, jnp.int32)
    y = lax.bitcast_convert_type(jnp.int32(0x5F3759DF) - (i >> 1), jnp.float32)
    for _ in range(3):
        y = y * (1.5 - 0.5 * x * y * y)
    return y


def _sc_body(q_hbm, mem_hbm, fx_hbm, bk_hbm, bv_hbm, ffn_hbm, img_hbm,
             out_hbm, rows_v, bias_v, wb_v, adap_v, qv_v, iv_v, lg_v):
    wid = lax.axis_index("s") * 2 + lax.axis_index("c")
    z = jnp.zeros((16,), jnp.float32)

    pltpu.sync_copy(q_hbm, qv_v)
    pltpu.sync_copy(img_hbm, iv_v)

    def _zero_pad_row(ch, carry):
        rows_v[_M + 1, pl.ds(ch * 16, 16)] = z
        return carry
    lax.fori_loop(0, _NCH, _zero_pad_row, 0)

    def _class_body(i, carry):
        cls = wid * _CPW + i

        @pl.when(cls < _C)
        def _():
            pltpu.sync_copy(mem_hbm.at[cls], rows_v.at[pl.ds(0, _M)])
            pltpu.sync_copy(fx_hbm.at[cls], rows_v.at[_M])
            pltpu.sync_copy(bk_hbm.at[cls], bias_v.at[0])
            pltpu.sync_copy(bv_hbm.at[cls], bias_v.at[1])
            pltpu.sync_copy(ffn_hbm.at[cls], bias_v.at[2])

            def _cc(ch, acc):
                a1, a2, a3 = acc
                sl = pl.ds(ch * 16, 16)
                qc = qv_v[sl]
                bkc = bias_v[0, sl]
                bvc = bias_v[1, sl]
                return (a1 + bkc * bkc, a2 + bvc * bvc, a3 + qc * bkc)
            bkbk, bvbv, qbk = lax.fori_loop(0, _NCH, _cc, (z, z, z))
            bkbk = _bsum(bkbk)
            bvbv = _bsum(bvbv)
            qbk = _bsum(qbk)

            wsum = z
            for r0, nr in _BLOCKS:
                def _p1(ch, acc):
                    sl = pl.ds(ch * 16, 16)
                    qc = qv_v[sl]
                    bkc = bias_v[0, sl]
                    bvc = bias_v[1, sl]
                    out = []
                    for j in range(nr):
                        rv = rows_v[r0 + j, sl]
                        aq, ab, av, ar, asm = acc[j]
                        out.append((aq + rv * qc, ab + rv * bkc,
                                    av + rv * bvc, ar + rv * rv, asm + rv))
                    return tuple(out)
                init = tuple((z, z, z, z, z) for _ in range(nr))
                res = lax.fori_loop(0, _NCH, _p1, init)
                for j in range(nr):
                    aq, ab, av, ar, asm = res[j]
                    qr = _bsum(aq)
                    rbk = _bsum(ab)
                    rbv = _bsum(av)
                    rr = _bsum(ar)
                    ss = _bsum(asm)
                    kk = rr + 2.0 * rbk + bkbk
                    vv = rr + 2.0 * rbv + bvbv
                    s = qr + qbk
                    sim = jnp.exp(_BETA * (s * _rsqrt(kk) - 1.0))
                    w = jnp.where(ss == 0.0, 0.0, sim * _rsqrt(vv))
                    wb_v[r0 + j, :] = w
                    wsum = wsum + w

            first = True
            for r0, nr in _BLOCKS:
                wvecs = [wb_v[r0 + j, :] for j in range(nr)]
                fi = first

                def _p2(ch, carry):
                    sl = pl.ds(ch * 16, 16)
                    acc = wsum * bias_v[1, sl] if fi else adap_v[sl]
                    for j in range(nr):
                        acc = acc + rows_v[r0 + j, sl] * wvecs[j]
                    adap_v[sl] = acc
                    return carry
                lax.fori_loop(0, _NCH, _p2, 0)
                first = False

            def _p3a(ch, acc):
                x = adap_v[pl.ds(ch * 16, 16)]
                return acc + x * x
            aa = _bsum(lax.fori_loop(0, _NCH, _p3a, z))
            r1 = _rsqrt(aa)

            def _p3b(ch, acc):
                a2, ai = acc
                sl = pl.ds(ch * 16, 16)
                x = adap_v[sl] * r1 + bias_v[2, sl]
                return (a2 + x * x, ai + x * iv_v[sl])
            aa2, ai = lax.fori_loop(0, _NCH, _p3b, (z, z))
            lg_v[i, :] = _LOGIT_SCALE * _bsum(ai) * _rsqrt(_bsum(aa2))
        return carry

    lax.fori_loop(0, _CPW, _class_body, 0)
    pltpu.sync_copy(lg_v, out_hbm.at[pl.ds(wid * _CPW, _CPW)])


def _q_body(img_ref, gb_ref, o_ref):
    s = jnp.sum(gb_ref[...], axis=0, keepdims=True) * (1.0 / _C) + img_ref[...]
    o_ref[...] = s * lax.rsqrt(jnp.sum(s * s))


def _softmax_body(x_ref, o_ref):
    x = x_ref[...]
    idx = lax.broadcasted_iota(jnp.int32, (8, 128), 0) * 128 + \
        lax.broadcasted_iota(jnp.int32, (8, 128), 1)
    x = jnp.where(idx < _C, x, -jnp.inf)
    e = jnp.where(idx < _C, jnp.exp(x - jnp.max(x)), 0.0)
    o_ref[...] = e / jnp.sum(e)


@jax.jit
def kernel(img_feat, image_feature_memory, fixed_global_feat_vanilla,
           global_bias, global_bias_key, global_bias_value, global_ffn_bias):
    q = pl.pallas_call(
        _q_body,
        out_shape=jax.ShapeDtypeStruct((1, _D), jnp.float32),
    )(img_feat, global_bias)

    sc = pl.kernel(
        _sc_body,
        mesh=plsc.VectorSubcoreMesh(core_axis_name="c", subcore_axis_name="s"),
        out_type=jax.ShapeDtypeStruct((_NW * _CPW, 16), jnp.float32),
        compiler_params=pltpu.CompilerParams(use_tc_tiling_on_sc=False,
                                            needs_layout_passes=False),
        scratch_types=[
            pltpu.VMEM((_RP, _D), jnp.float32),
            pltpu.VMEM((3, _D), jnp.float32),
            pltpu.VMEM((_RP, 16), jnp.float32),
            pltpu.VMEM((_D,), jnp.float32),
            pltpu.VMEM((_D,), jnp.float32),
            pltpu.VMEM((_D,), jnp.float32),
            pltpu.VMEM((_CPW, 16), jnp.float32),
        ],
    )
    lg16 = sc(q.reshape(_D), image_feature_memory,
              fixed_global_feat_vanilla.reshape(_C, _D), global_bias_key,
              global_bias_value, global_ffn_bias, img_feat.reshape(_D))

    probs = pl.pallas_call(
        _softmax_body,
        out_shape=jax.ShapeDtypeStruct((8, 128), jnp.float32),
    )(lg16[:, 0].reshape(8, 128))
    return probs.reshape(_NW * _CPW)[:_C][None, :]


# speedup vs baseline: 1.0309x; 1.0309x over previous
import functools

import jax
import jax.numpy as jnp
from jax import lax
from jax.experimental import pallas as pl
from jax.experimental.pallas import tpu as pltpu
from jax.experimental.pallas import tpu_sc as plsc

_BETA = 5.5
_LOGIT_SCALE = 100.0
_C, _M, _D = 1000, 50, 1024
_NW = 32
_CPW = 32
_NCH = _D // 16


def _bsum(v):
    return jnp.broadcast_to(jnp.sum(v), (16,))


def _rsqrt(x):
    i = lax.bitcast_convert_type(x, jnp.int32)
    y = lax.bitcast_convert_type(jnp.int32(0x5F3759DF) - (i >> 1), jnp.float32)
    for _ in range(3):
        y = y * (1.5 - 0.5 * x * y * y)
    return y


def _sc_body(q_hbm, mem_hbm, aux_hbm, img_hbm,
             out_hbm, rows_v, aux_v, adap_v, qv_v, iv_v, lg_v, sem_a, sem_b):
    wid = lax.axis_index("s") * 2 + lax.axis_index("c")
    z = jnp.zeros((16,), jnp.float32)
    base = wid * _CPW

    pltpu.sync_copy(q_hbm, qv_v)
    pltpu.sync_copy(img_hbm, iv_v)
    pltpu.async_copy(mem_hbm.at[base], rows_v.at[0], sem_a)
    pltpu.async_copy(aux_hbm.at[base], aux_v.at[0], sem_a)

    def _row_weight(accs, bkbk, bvbv, qbk):
        aq, ab, av, ar, asm = accs
        qr = _bsum(aq)
        rbk = _bsum(ab)
        rbv = _bsum(av)
        rr = _bsum(ar)
        ss = _bsum(asm)
        kk = rr + 2.0 * rbk + bkbk
        vv = rr + 2.0 * rbv + bvbv
        s = qr + qbk
        sim = jnp.exp(_BETA * (s * _rsqrt(kk) - 1.0))
        return jnp.where(ss == 0.0, 0.0, sim * _rsqrt(vv))

    def _compute(b, k_idx):
        def _cc(ch, acc):
            a1, a2, a3 = acc
            sl = pl.ds(ch * 16, 16)
            qc = qv_v[sl]
            bkc = aux_v[b, 1, sl]
            bvc = aux_v[b, 2, sl]
            return (a1 + bkc * bkc, a2 + bvc * bvc, a3 + qc * bkc)
        bkbk, bvbv, qbk = lax.fori_loop(0, _NCH, _cc, (z, z, z), unroll=2)
        bkbk = _bsum(bkbk)
        bvbv = _bsum(bvbv)
        qbk = _bsum(qbk)

        lrefs = [(rows_v, 48), (rows_v, 49), (aux_v, 0)]

        def _p1l(ch, acc):
            sl = pl.ds(ch * 16, 16)
            qc = qv_v[sl]
            bkc = aux_v[b, 1, sl]
            bvc = aux_v[b, 2, sl]
            out = []
            for j, (ref, ri) in enumerate(lrefs):
                rv = ref[b, ri, sl]
                aq, ab, av, ar, asm = acc[j]
                out.append((aq + rv * qc, ab + rv * bkc, av + rv * bvc,
                            ar + rv * rv, asm + rv))
            return tuple(out)
        res = lax.fori_loop(0, _NCH, _p1l, tuple((z, z, z, z, z)
                                                 for _ in lrefs), unroll=2)
        lw = [_row_weight(res[j], bkbk, bvbv, qbk) for j in range(3)]
        wsum = lw[0] + lw[1] + lw[2]

        def _p2l(ch, carry):
            sl = pl.ds(ch * 16, 16)
            adap_v[sl] = (rows_v[b, 48, sl] * lw[0] +
                          rows_v[b, 49, sl] * lw[1] + aux_v[b, 0, sl] * lw[2])
            return carry
        lax.fori_loop(0, _NCH, _p2l, 0, unroll=2)

        def _blk(bi, wsum):
            r0 = bi * 8

            def _p1(ch, acc):
                sl = pl.ds(ch * 16, 16)
                qc = qv_v[sl]
                bkc = aux_v[b, 1, sl]
                bvc = aux_v[b, 2, sl]
                out = []
                for j in range(8):
                    rv = rows_v[b, r0 + j, sl]
                    aq, ab, av, ar, asm = acc[j]
                    out.append((aq + rv * qc, ab + rv * bkc, av + rv * bvc,
                                ar + rv * rv, asm + rv))
                return tuple(out)
            res = lax.fori_loop(0, _NCH, _p1, tuple((z, z, z, z, z)
                                                    for _ in range(8)),
                                unroll=2)
            ws = [_row_weight(res[j], bkbk, bvbv, qbk) for j in range(8)]
            for w in ws:
                wsum = wsum + w

            def _p2(ch, carry):
                sl = pl.ds(ch * 16, 16)
                acc = adap_v[sl]
                for j in range(8):
                    acc = acc + rows_v[b, r0 + j, sl] * ws[j]
                adap_v[sl] = acc
                return carry
            lax.fori_loop(0, _NCH, _p2, 0, unroll=2)
            return wsum
        wsum = lax.fori_loop(0, 6, _blk, wsum)

        def _p3a(ch, acc):
            sl = pl.ds(ch * 16, 16)
            x = adap_v[sl] + wsum * aux_v[b, 2, sl]
            return acc + x * x
        aa = _bsum(lax.fori_loop(0, _NCH, _p3a, z, unroll=2))
        r1 = _rsqrt(aa)

        def _p3b(ch, acc):
            a2, ai = acc
            sl = pl.ds(ch * 16, 16)
            x = (adap_v[sl] + wsum * aux_v[b, 2, sl]) * r1 + aux_v[b, 3, sl]
            return (a2 + x * x, ai + x * iv_v[sl])
        aa2, ai = lax.fori_loop(0, _NCH, _p3b, (z, z), unroll=2)
        lg_v[k_idx, :] = _LOGIT_SCALE * _bsum(ai) * _rsqrt(_bsum(aa2))

    def _do_class(b, k_idx, sem):
        cls = base + k_idx
        nxt = cls + 1

        @pl.when((k_idx + 1 < _CPW) & (nxt < _C))
        def _():
            osem = sem_b if b == 0 else sem_a
            pltpu.async_copy(mem_hbm.at[nxt], rows_v.at[1 - b], osem)
            pltpu.async_copy(aux_hbm.at[nxt], aux_v.at[1 - b], osem)

        @pl.when(cls < _C)
        def _():
            pltpu.make_async_copy(mem_hbm.at[cls], rows_v.at[b], sem).wait()
            pltpu.make_async_copy(aux_hbm.at[cls], aux_v.at[b], sem).wait()
            _compute(b, k_idx)

    def _class_pair(i, carry):
        _do_class(0, 2 * i, sem_a)
        _do_class(1, 2 * i + 1, sem_b)
        return carry
    lax.fori_loop(0, _CPW // 2, _class_pair, 0)
    pltpu.sync_copy(lg_v, out_hbm.at[pl.ds(base, _CPW)])


def _q_body(img_ref, gb_ref, o_ref):
    s = jnp.sum(gb_ref[...], axis=0, keepdims=True) * (1.0 / _C) + img_ref[...]
    o_ref[...] = s * lax.rsqrt(jnp.sum(s * s))


def _softmax_body(x_ref, o_ref):
    x = x_ref[...]
    idx = lax.broadcasted_iota(jnp.int32, (8, 128), 0) * 128 + \
        lax.broadcasted_iota(jnp.int32, (8, 128), 1)
    x = jnp.where(idx < _C, x, -jnp.inf)
    e = jnp.where(idx < _C, jnp.exp(x - jnp.max(x)), 0.0)
    o_ref[...] = e / jnp.sum(e)


@jax.jit
def kernel(img_feat, image_feature_memory, fixed_global_feat_vanilla,
           global_bias, global_bias_key, global_bias_value, global_ffn_bias):
    q = pl.pallas_call(
        _q_body,
        out_shape=jax.ShapeDtypeStruct((1, _D), jnp.float32),
    )(img_feat, global_bias)

    aux = jnp.stack([fixed_global_feat_vanilla.reshape(_C, _D),
                     global_bias_key, global_bias_value, global_ffn_bias],
                    axis=1)

    sc = pl.kernel(
        _sc_body,
        mesh=plsc.VectorSubcoreMesh(core_axis_name="c", subcore_axis_name="s"),
        out_type=jax.ShapeDtypeStruct((_NW * _CPW, 16), jnp.float32),
        compiler_params=pltpu.CompilerParams(use_tc_tiling_on_sc=False,
                                             needs_layout_passes=False),
        scratch_types=[
            pltpu.VMEM((2, _M, _D), jnp.float32),
            pltpu.VMEM((2, 4, _D), jnp.float32),
            pltpu.VMEM((_D,), jnp.float32),
            pltpu.VMEM((_D,), jnp.float32),
            pltpu.VMEM((_D,), jnp.float32),
            pltpu.VMEM((_CPW, 16), jnp.float32),
            pltpu.SemaphoreType.DMA,
            pltpu.SemaphoreType.DMA,
        ],
    )
    lg16 = sc(q.reshape(_D), image_feature_memory, aux, img_feat.reshape(_D))

    probs = pl.pallas_call(
        _softmax_body,
        out_shape=jax.ShapeDtypeStruct((8, 128), jnp.float32),
    )(lg16[:, 0].reshape(8, 128))
    return probs.reshape(_NW * _CPW)[:_C][None, :]


# speedup vs baseline: 3.2285x; 3.1319x over previous
import functools

import jax
import jax.numpy as jnp
from jax import lax
from jax.experimental import pallas as pl
from jax.experimental.pallas import tpu as pltpu
from jax.experimental.pallas import tpu_sc as plsc

_BETA = 5.5
_LOGIT_SCALE = 100.0
_C, _M, _D = 1000, 50, 1024
_NW = 32
_S = 320
_CPW = _S // _NW
_NCH = _D // 16
_TCB = 8


def _bsum(v):
    return jnp.broadcast_to(jnp.sum(v), (16,))


def _rsqrt(x):
    i = lax.bitcast_convert_type(x, jnp.int32)
    y = lax.bitcast_convert_type(jnp.int32(0x5F3759DF) - (i >> 1), jnp.float32)
    for _ in range(3):
        y = y * (1.5 - 0.5 * x * y * y)
    return y


def _sc_body(q_hbm, mem_hbm, fx_hbm, bk_hbm, bv_hbm, ffn_hbm, img_hbm,
             out_hbm, rows_v, aux_v, adap_v, qv_v, iv_v, lg_v, sem_a, sem_b):
    wid = lax.axis_index("s") * 2 + lax.axis_index("c")
    z = jnp.zeros((16,), jnp.float32)
    base = wid * _CPW

    pltpu.sync_copy(q_hbm, qv_v)
    pltpu.sync_copy(img_hbm, iv_v)
    pltpu.async_copy(mem_hbm.at[base], rows_v.at[0], sem_a)
    for _i, _src in enumerate((fx_hbm, bk_hbm, bv_hbm, ffn_hbm)):
        pltpu.async_copy(_src.at[base], aux_v.at[0, _i], sem_a)

    def _row_weight(accs, bkbk, bvbv, qbk):
        aq, ab, av, ar, asm = accs
        rr = _bsum(ar)
        kk = rr + 2.0 * _bsum(ab) + bkbk
        vv = rr + 2.0 * _bsum(av) + bvbv
        s = _bsum(aq) + qbk
        sim = jnp.exp(_BETA * (s * _rsqrt(kk) - 1.0))
        return jnp.where(_bsum(asm) == 0.0, 0.0, sim * _rsqrt(vv))

    def _compute(b, k_idx):
        def _cc(ch, acc):
            a1, a2, a3 = acc
            sl = pl.ds(ch * 16, 16)
            qc = qv_v[sl]
            bkc = aux_v[b, 1, sl]
            bvc = aux_v[b, 2, sl]
            return (a1 + bkc * bkc, a2 + bvc * bvc, a3 + qc * bkc)
        bkbk, bvbv, qbk = lax.fori_loop(0, _NCH, _cc, (z, z, z), unroll=2)
        bkbk = _bsum(bkbk)
        bvbv = _bsum(bvbv)
        qbk = _bsum(qbk)

        def _p1_block(loads):
            nr = len(loads)

            def _p1(ch, acc):
                sl = pl.ds(ch * 16, 16)
                qc = qv_v[sl]
                bkc = aux_v[b, 1, sl]
                bvc = aux_v[b, 2, sl]
                out = []
                for j in range(nr):
                    rv = loads[j](sl)
                    aq, ab, av, ar, asm = acc[j]
                    out.append((aq + rv * qc, ab + rv * bkc, av + rv * bvc,
                                ar + rv * rv, asm + rv))
                return tuple(out)
            res = lax.fori_loop(0, _NCH, _p1, tuple((z, z, z, z, z)
                                                    for _ in range(nr)),
                                unroll=2)
            return [_row_weight(res[j], bkbk, bvbv, qbk) for j in range(nr)]

        lloads = [lambda sl: rows_v[b, 48, sl], lambda sl: rows_v[b, 49, sl],
                  lambda sl: aux_v[b, 0, sl]]
        lw = _p1_block(lloads)
        wsum = lw[0] + lw[1] + lw[2]

        def _p2l(ch, carry):
            sl = pl.ds(ch * 16, 16)
            adap_v[sl] = (rows_v[b, 48, sl] * lw[0] +
                          rows_v[b, 49, sl] * lw[1] + aux_v[b, 0, sl] * lw[2])
            return carry
        lax.fori_loop(0, _NCH, _p2l, 0, unroll=2)

        for r0 in range(0, 48, 6):
            ws = _p1_block([(lambda sl, r=r0 + j: rows_v[b, r, sl])
                            for j in range(6)])
            for w in ws:
                wsum = wsum + w

            def _p2(ch, carry, r0=r0, ws=ws):
                sl = pl.ds(ch * 16, 16)
                acc = adap_v[sl]
                for j in range(6):
                    acc = acc + rows_v[b, r0 + j, sl] * ws[j]
                adap_v[sl] = acc
                return carry
            lax.fori_loop(0, _NCH, _p2, 0, unroll=2)

        def _p3a(ch, acc):
            sl = pl.ds(ch * 16, 16)
            x = adap_v[sl] + wsum * aux_v[b, 2, sl]
            return acc + x * x
        aa = _bsum(lax.fori_loop(0, _NCH, _p3a, z, unroll=2))
        r1 = _rsqrt(aa)

        def _p3b(ch, acc):
            a2, ai = acc
            sl = pl.ds(ch * 16, 16)
            x = (adap_v[sl] + wsum * aux_v[b, 2, sl]) * r1 + aux_v[b, 3, sl]
            return (a2 + x * x, ai + x * iv_v[sl])
        aa2, ai = lax.fori_loop(0, _NCH, _p3b, (z, z), unroll=2)
        lg_v[k_idx, :] = _LOGIT_SCALE * _bsum(ai) * _rsqrt(_bsum(aa2))

    def _do_class(b, k_idx, sem):
        cls = base + k_idx

        @pl.when(k_idx + 1 < _CPW)
        def _():
            osem = sem_b if b == 0 else sem_a
            pltpu.async_copy(mem_hbm.at[cls + 1], rows_v.at[1 - b], osem)
            for _i, _src in enumerate((fx_hbm, bk_hbm, bv_hbm, ffn_hbm)):
                pltpu.async_copy(_src.at[cls + 1], aux_v.at[1 - b, _i], osem)

        pltpu.make_async_copy(mem_hbm.at[cls], rows_v.at[b], sem).wait()
        for _i, _src in enumerate((fx_hbm, bk_hbm, bv_hbm, ffn_hbm)):
            pltpu.make_async_copy(_src.at[cls], aux_v.at[b, _i], sem).wait()
        _compute(b, k_idx)

    def _class_pair(i, carry):
        _do_class(0, 2 * i, sem_a)
        _do_class(1, 2 * i + 1, sem_b)
        return carry
    lax.fori_loop(0, _CPW // 2, _class_pair, 0)
    pltpu.sync_copy(lg_v, out_hbm.at[pl.ds(base, _CPW)])


def _tc_body(q_ref, img_ref, mem_ref, fx_ref, bk_ref, bv_ref, fn_ref, o_ref):
    q = q_ref[...][None]
    img = img_ref[...][None]
    mem = mem_ref[...]
    fx = fx_ref[...]
    bk = bk_ref[...]
    bv = bv_ref[...]
    fn = fn_ref[...]
    bkbk = jnp.sum(bk * bk, -1, keepdims=True)
    bvbv = jnp.sum(bv * bv, -1, keepdims=True)
    qbk = jnp.sum(bk * q, -1, keepdims=True)

    def _w(r):
        rr = jnp.sum(r * r, -1, keepdims=True)
        kk = rr + 2.0 * jnp.sum(r * bk, -1, keepdims=True) + bkbk
        vv = rr + 2.0 * jnp.sum(r * bv, -1, keepdims=True) + bvbv
        s = jnp.sum(r * q, -1, keepdims=True) + qbk
        sim = jnp.exp(_BETA * (s * lax.rsqrt(kk) - 1.0))
        empty = jnp.sum(r, -1, keepdims=True) == 0.0
        return jnp.where(empty, 0.0, sim * lax.rsqrt(vv))

    w = _w(mem)
    w_f = _w(fx)
    a = jnp.sum(mem * w, 1, keepdims=True) + w_f * fx + \
        (jnp.sum(w, 1, keepdims=True) + w_f) * bv
    a = a * lax.rsqrt(jnp.sum(a * a, -1, keepdims=True))
    a2 = a + fn
    o_ref[...] = _LOGIT_SCALE * jnp.sum(a2 * img, -1, keepdims=True) * \
        lax.rsqrt(jnp.sum(a2 * a2, -1, keepdims=True))


def _q_body(img_ref, gb_ref, o_ref):
    s = jnp.sum(gb_ref[...], axis=0, keepdims=True) * (1.0 / _C) + img_ref[...]
    o_ref[...] = s * lax.rsqrt(jnp.sum(s * s))


def _softmax_body(x_ref, o_ref):
    x = x_ref[...]
    idx = lax.broadcasted_iota(jnp.int32, (8, 128), 0) * 128 + \
        lax.broadcasted_iota(jnp.int32, (8, 128), 1)
    x = jnp.where(idx < _C, x, -jnp.inf)
    e = jnp.where(idx < _C, jnp.exp(x - jnp.max(x)), 0.0)
    o_ref[...] = e / jnp.sum(e)


@jax.jit
def kernel(img_feat, image_feature_memory, fixed_global_feat_vanilla,
           global_bias, global_bias_key, global_bias_value, global_ffn_bias):
    q = pl.pallas_call(
        _q_body,
        out_shape=jax.ShapeDtypeStruct((1, _D), jnp.float32),
    )(img_feat, global_bias)

    sc = pl.kernel(
        _sc_body,
        mesh=plsc.VectorSubcoreMesh(core_axis_name="c", subcore_axis_name="s"),
        out_type=jax.ShapeDtypeStruct((_S, 16), jnp.float32),
        compiler_params=pltpu.CompilerParams(use_tc_tiling_on_sc=False,
                                             needs_layout_passes=False),
        scratch_types=[
            pltpu.VMEM((2, _M, _D), jnp.float32),
            pltpu.VMEM((2, 4, _D), jnp.float32),
            pltpu.VMEM((_D,), jnp.float32),
            pltpu.VMEM((_D,), jnp.float32),
            pltpu.VMEM((_D,), jnp.float32),
            pltpu.VMEM((_CPW, 16), jnp.float32),
            pltpu.SemaphoreType.DMA,
            pltpu.SemaphoreType.DMA,
        ],
    )
    sc_lg16 = sc(q.reshape(_D), image_feature_memory,
                 fixed_global_feat_vanilla.reshape(_C, _D),
                 global_bias_key, global_bias_value, global_ffn_bias,
                 img_feat.reshape(_D))

    ntc = (_C - _S) // _TCB
    tc_lg = pl.pallas_call(
        _tc_body,
        grid=(ntc,),
        in_specs=[
            pl.BlockSpec((1, _D), lambda i: (0, 0)),
            pl.BlockSpec((1, _D), lambda i: (0, 0)),
            pl.BlockSpec((_TCB, _M, _D), lambda i: (_S // _TCB + i, 0, 0)),
            pl.BlockSpec((_TCB, 1, _D), lambda i: (_S // _TCB + i, 0, 0)),
            pl.BlockSpec((_TCB, 1, _D), lambda i: (_S // _TCB + i, 0, 0)),
            pl.BlockSpec((_TCB, 1, _D), lambda i: (_S // _TCB + i, 0, 0)),
            pl.BlockSpec((_TCB, 1, _D), lambda i: (_S // _TCB + i, 0, 0)),
        ],
        out_specs=pl.BlockSpec((_TCB, 1, 1), lambda i: (i, 0, 0)),
        out_shape=jax.ShapeDtypeStruct((ntc * _TCB, 1, 1), jnp.float32),
    )(q, img_feat, image_feature_memory, fixed_global_feat_vanilla,
      global_bias_key.reshape(_C, 1, _D),
      global_bias_value.reshape(_C, 1, _D),
      global_ffn_bias.reshape(_C, 1, _D))

    lg = jnp.concatenate([sc_lg16[:, 0], tc_lg.reshape(_C - _S),
                          jnp.zeros(1024 - _C, jnp.float32)])
    probs = pl.pallas_call(
        _softmax_body,
        out_shape=jax.ShapeDtypeStruct((8, 128), jnp.float32),
    )(lg.reshape(8, 128))
    return probs.reshape(1024)[:_C][None, :]


# speedup vs baseline: 5.3774x; 1.6656x over previous
import functools

import jax
import jax.numpy as jnp
from jax import lax
from jax.experimental import pallas as pl
from jax.experimental.pallas import tpu as pltpu
from jax.experimental.pallas import tpu_sc as plsc

_BETA = 5.5
_LOGIT_SCALE = 100.0
_C, _M, _D = 1000, 50, 1024
_NW = 32
_CPW = 8
_S = _NW * _CPW
_NCH = _D // 16
_TCB = 8


def _bsum(v):
    return jnp.broadcast_to(jnp.sum(v), (16,))


def _rsqrt(x):
    i = lax.bitcast_convert_type(x, jnp.int32)
    y = lax.bitcast_convert_type(jnp.int32(0x5F3759DF) - (i >> 1), jnp.float32)
    for _ in range(3):
        y = y * (1.5 - 0.5 * x * y * y)
    return y


def _sc_body(q_hbm, mem_hbm, fx_hbm, bk_hbm, bv_hbm, ffn_hbm, img_hbm,
             out_hbm, rows_v, bkv, bvv, fnv, fxv, adap_v, qv_v, iv_v, lg_v,
             sem_a):
    wid = lax.axis_index("s") * 2 + lax.axis_index("c")
    z = jnp.zeros((16,), jnp.float32)
    base = wid * _CPW

    pltpu.sync_copy(q_hbm, qv_v)
    pltpu.sync_copy(img_hbm, iv_v)
    pltpu.sync_copy(bk_hbm.at[pl.ds(base, _CPW)], bkv)
    pltpu.sync_copy(bv_hbm.at[pl.ds(base, _CPW)], bvv)
    pltpu.sync_copy(ffn_hbm.at[pl.ds(base, _CPW)], fnv)
    pltpu.sync_copy(fx_hbm.at[pl.ds(base, _CPW)], fxv)

    def _row_weight(accs, bkbk, bvbv, qbk):
        aq, ab, av, ar, asm = accs
        rr = _bsum(ar)
        kk = rr + 2.0 * _bsum(ab) + bkbk
        vv = rr + 2.0 * _bsum(av) + bvbv
        s = _bsum(aq) + qbk
        sim = jnp.exp(_BETA * (s * _rsqrt(kk) - 1.0))
        return jnp.where(_bsum(asm) == 0.0, 0.0, sim * _rsqrt(vv))

    def _compute(k_idx):
        def _cc(ch, acc):
            a1, a2, a3 = acc
            sl = pl.ds(ch * 16, 16)
            qc = qv_v[0, sl]
            bkc = bkv[k_idx, sl]
            bvc = bvv[k_idx, sl]
            return (a1 + bkc * bkc, a2 + bvc * bvc, a3 + qc * bkc)
        bkbk, bvbv, qbk = lax.fori_loop(0, _NCH, _cc, (z, z, z), unroll=2)
        bkbk = _bsum(bkbk)
        bvbv = _bsum(bvbv)
        qbk = _bsum(qbk)

        def _p1_block(loads):
            nr = len(loads)

            def _p1(ch, acc):
                sl = pl.ds(ch * 16, 16)
                qc = qv_v[0, sl]
                bkc = bkv[k_idx, sl]
                bvc = bvv[k_idx, sl]
                out = []
                for j in range(nr):
                    rv = loads[j](sl)
                    aq, ab, av, ar, asm = acc[j]
                    out.append((aq + rv * qc, ab + rv * bkc, av + rv * bvc,
                                ar + rv * rv, asm + rv))
                return tuple(out)
            res = lax.fori_loop(0, _NCH, _p1, tuple((z, z, z, z, z)
                                                    for _ in range(nr)),
                                unroll=2)
            return [_row_weight(res[j], bkbk, bvbv, qbk) for j in range(nr)]

        lloads = [lambda sl: rows_v[48, sl], lambda sl: rows_v[49, sl],
                  lambda sl: fxv[k_idx, 0, sl]]
        lw = _p1_block(lloads)
        wsum = lw[0] + lw[1] + lw[2]

        def _p2l(ch, carry):
            sl = pl.ds(ch * 16, 16)
            adap_v[sl] = (rows_v[48, sl] * lw[0] +
                          rows_v[49, sl] * lw[1] + fxv[k_idx, 0, sl] * lw[2])
            return carry
        lax.fori_loop(0, _NCH, _p2l, 0, unroll=2)

        for r0 in range(0, 48, 6):
            ws = _p1_block([(lambda sl, r=r0 + j: rows_v[r, sl])
                            for j in range(6)])
            for w in ws:
                wsum = wsum + w

            def _p2(ch, carry, r0=r0, ws=ws):
                sl = pl.ds(ch * 16, 16)
                acc = adap_v[sl]
                for j in range(6):
                    acc = acc + rows_v[r0 + j, sl] * ws[j]
                adap_v[sl] = acc
                return carry
            lax.fori_loop(0, _NCH, _p2, 0, unroll=2)

        def _p3a(ch, acc):
            sl = pl.ds(ch * 16, 16)
            x = adap_v[sl] + wsum * bvv[k_idx, sl]
            return acc + x * x
        aa = _bsum(lax.fori_loop(0, _NCH, _p3a, z, unroll=2))
        r1 = _rsqrt(aa)

        def _p3b(ch, acc):
            a2, ai = acc
            sl = pl.ds(ch * 16, 16)
            x = (adap_v[sl] + wsum * bvv[k_idx, sl]) * r1 + fnv[k_idx, sl]
            return (a2 + x * x, ai + x * iv_v[0, sl])
        aa2, ai = lax.fori_loop(0, _NCH, _p3b, (z, z), unroll=2)
        lg_v[k_idx, :] = _LOGIT_SCALE * _bsum(ai) * _rsqrt(_bsum(aa2))

    def _class_body(k_idx, carry):
        pltpu.async_copy(mem_hbm.at[base + k_idx], rows_v, sem_a).wait()
        _compute(k_idx)
        return carry
    lax.fori_loop(0, _CPW, _class_body, 0)
    pltpu.sync_copy(lg_v, out_hbm.at[pl.ds(base, _CPW)])


def _tc_body(q_ref, img_ref, mem_ref, fx_ref, bk_ref, bv_ref, fn_ref, o_ref):
    mem = mem_ref[...]
    fx = fx_ref[...]
    bk = bk_ref[...]
    bv = bv_ref[...]
    fn = fn_ref[...]

    def _b3(x, shape, dims):
        return lax.broadcast_in_dim(x, shape, dims)

    q3 = _b3(q_ref[...], (_TCB, _M, _D), (0, 2))
    bk3 = _b3(bk, (_TCB, _M, _D), (0, 2))
    bv3 = _b3(bv, (_TCB, _M, _D), (0, 2))
    qf3 = _b3(q_ref[...], (_TCB, 1, _D), (0, 2))
    bkf3 = _b3(bk, (_TCB, 1, _D), (0, 2))
    bvf3 = _b3(bv, (_TCB, 1, _D), (0, 2))
    bkbk = jnp.sum(bk * bk, -1, keepdims=True)
    bvbv = jnp.sum(bv * bv, -1, keepdims=True)
    qbk = jnp.sum(bk * _b3(q_ref[...], (_TCB, _D), (0, 1)), -1, keepdims=True)

    def _w(r, qx, bkx, bvx):
        rr = jnp.sum(r * r, -1)
        kk = rr + 2.0 * jnp.sum(r * bkx, -1) + bkbk
        vv = rr + 2.0 * jnp.sum(r * bvx, -1) + bvbv
        s = jnp.sum(r * qx, -1) + qbk
        sim = jnp.exp(_BETA * (s * lax.rsqrt(kk) - 1.0))
        empty = jnp.sum(r, -1) == 0.0
        return jnp.where(empty, 0.0, sim * lax.rsqrt(vv))

    w = _w(mem, q3, bk3, bv3)
    w_f = _w(fx, qf3, bkf3, bvf3)
    a = jnp.sum(mem * _b3(w, (_TCB, _M, _D), (0, 1)), 1) + \
        jnp.sum(fx * _b3(w_f, (_TCB, 1, _D), (0, 1)), 1) + \
        (jnp.sum(w, -1, keepdims=True) + w_f) * bv
    a = a * lax.rsqrt(jnp.sum(a * a, -1, keepdims=True))
    a2 = a + fn
    img2 = _b3(img_ref[...], (_TCB, _D), (0, 1))
    o_ref[...] = _LOGIT_SCALE * jnp.sum(a2 * img2, -1, keepdims=True) * \
        lax.rsqrt(jnp.sum(a2 * a2, -1, keepdims=True))


def _q_body(img_ref, gb_ref, o_ref):
    s = jnp.sum(gb_ref[...], axis=0, keepdims=True) * (1.0 / _C) + img_ref[...]
    o_ref[...] = s * lax.rsqrt(jnp.sum(s * s))


def _softmax_body(x_ref, o_ref):
    x = x_ref[...]
    idx = lax.broadcasted_iota(jnp.int32, (8, 128), 0) * 128 + \
        lax.broadcasted_iota(jnp.int32, (8, 128), 1)
    x = jnp.where(idx < _C, x, -jnp.inf)
    e = jnp.where(idx < _C, jnp.exp(x - jnp.max(x)), 0.0)
    o_ref[...] = e / jnp.sum(e)


@jax.jit
def kernel(img_feat, image_feature_memory, fixed_global_feat_vanilla,
           global_bias, global_bias_key, global_bias_value, global_ffn_bias):
    q = pl.pallas_call(
        _q_body,
        out_shape=jax.ShapeDtypeStruct((1, _D), jnp.float32),
    )(img_feat, global_bias)

    sc = pl.kernel(
        _sc_body,
        mesh=plsc.VectorSubcoreMesh(core_axis_name="c", subcore_axis_name="s"),
        out_type=jax.ShapeDtypeStruct((_S, 16), jnp.float32),
        compiler_params=pltpu.CompilerParams(use_tc_tiling_on_sc=True,
                                             needs_layout_passes=False),
        scratch_types=[
            pltpu.VMEM((_M, _D), jnp.float32),
            pltpu.VMEM((_CPW, _D), jnp.float32),
            pltpu.VMEM((_CPW, _D), jnp.float32),
            pltpu.VMEM((_CPW, _D), jnp.float32),
            pltpu.VMEM((_CPW, 1, _D), jnp.float32),
            pltpu.VMEM((_D,), jnp.float32),
            pltpu.VMEM((1, _D), jnp.float32),
            pltpu.VMEM((1, _D), jnp.float32),
            pltpu.VMEM((_CPW, 16), jnp.float32),
            pltpu.SemaphoreType.DMA,
        ],
    )
    sc_lg16 = sc(q, image_feature_memory, fixed_global_feat_vanilla,
                 global_bias_key, global_bias_value, global_ffn_bias,
                 img_feat)

    ntc = (_C - _S) // _TCB
    tc_lg = pl.pallas_call(
        _tc_body,
        grid=(ntc,),
        in_specs=[
            pl.BlockSpec((1, _D), lambda i: (0, 0)),
            pl.BlockSpec((1, _D), lambda i: (0, 0)),
            pl.BlockSpec((_TCB, _M, _D), lambda i: (_S // _TCB + i, 0, 0)),
            pl.BlockSpec((_TCB, 1, _D), lambda i: (_S // _TCB + i, 0, 0)),
            pl.BlockSpec((_TCB, _D), lambda i: (_S // _TCB + i, 0)),
            pl.BlockSpec((_TCB, _D), lambda i: (_S // _TCB + i, 0)),
            pl.BlockSpec((_TCB, _D), lambda i: (_S // _TCB + i, 0)),
        ],
        out_specs=pl.BlockSpec((_TCB, 1), lambda i: (i, 0)),
        out_shape=jax.ShapeDtypeStruct((_C - _S, 1), jnp.float32),
    )(q, img_feat, image_feature_memory, fixed_global_feat_vanilla,
      global_bias_key, global_bias_value, global_ffn_bias)

    lg = jnp.concatenate([sc_lg16[:, 0], tc_lg[:, 0],
                          jnp.zeros(1024 - _C, jnp.float32)])
    probs = pl.pallas_call(
        _softmax_body,
        out_shape=jax.ShapeDtypeStruct((8, 128), jnp.float32),
    )(lg.reshape(8, 128))
    return probs.reshape(1024)[:_C][None, :]
